# baseline (speedup 1.0000x reference)
import jax
import jax.numpy as jnp
from jax import lax
from jax.experimental import pallas as pl
from jax.experimental.pallas import tpu as pltpu
from jax.experimental.pallas import tpu_sc as plsc

_D = 128
_NROWS = 12
_S1 = 128 * 64
_S2 = 64
_O2 = 4
_O3 = 7
_NIDX = 24


def _body(idx_hbm, xflat_hbm, out_hbm, pack_v, idx_v, rows_v, sem):
    pltpu.sync_copy(idx_hbm, pack_v)
    lane = lax.iota(jnp.int32, 16)
    three = jnp.full((16,), 3, jnp.int32)
    four = jnp.full((16,), 4, jnp.int32)
    twelve = jnp.full((16,), 12, jnp.int32)
    r = lax.rem(lax.div(lane, three), four)
    c = lax.rem(lane, three) + _O2
    l3 = lax.rem(lane, twelve) + _O3
    a = plsc.load_gather(pack_v, [r])
    b = plsc.load_gather(pack_v, [c])
    g = plsc.load_gather(pack_v, [l3])
    idx_v[...] = a * _S1 + b * _S2 + g
    pltpu.async_copy(xflat_hbm.at[idx_v], rows_v, sem).wait()
    pltpu.sync_copy(rows_v.at[pl.ds(0, _NROWS)], out_hbm)


def kernel(x, index1, index2, index3):
    xflat = x.reshape(-1, _D)
    idx_packed = jnp.zeros((_NIDX,), jnp.int32)
    idx_packed = lax.dynamic_update_slice(idx_packed, index1.reshape(4), (0,))
    idx_packed = lax.dynamic_update_slice(idx_packed, index2.reshape(3), (_O2,))
    idx_packed = lax.dynamic_update_slice(idx_packed, index3.reshape(_NROWS), (_O3,))
    mesh = plsc.VectorSubcoreMesh(
        core_axis_name="c", subcore_axis_name="s", num_cores=1, num_subcores=1)
    out = pl.kernel(
        _body,
        out_type=jax.ShapeDtypeStruct((_NROWS, _D), jnp.float32),
        mesh=mesh,
        compiler_params=pltpu.CompilerParams(
            needs_layout_passes=False,
            skip_device_barrier=True,
            disable_bounds_checks=True,
            disable_semaphore_checks=True,
        ),
        scratch_types=[
            pltpu.VMEM((_NIDX,), jnp.int32),
            pltpu.VMEM((16,), jnp.int32),
            pltpu.VMEM((16, _D), jnp.float32),
            pltpu.SemaphoreType.DMA,
        ],
    )(idx_packed, xflat)
    return out.reshape(4, 3, _D)


# speedup vs baseline: 1.0345x; 1.0345x over previous
import jax
import jax.numpy as jnp
from jax import lax
from jax.experimental import pallas as pl
from jax.experimental.pallas import tpu as pltpu
from jax.experimental.pallas import tpu_sc as plsc

_D = 128
_NROWS = 12
_S1 = 128 * 64
_S2 = 64
_O2 = 4
_O3 = 7
_NIDX = 24


def _body(idx_hbm, xflat_hbm, out_hbm, idx_s, sem):
    pltpu.sync_copy(idx_hbm, idx_s)
    copies = []
    for i in range(_NROWS):
        a = idx_s[i // 3]
        b = idx_s[_O2 + i % 3]
        g = idx_s[_O3 + i]
        flat = a * _S1 + b * _S2 + g
        cp = pltpu.make_async_copy(
            xflat_hbm.at[pl.ds(flat, 1)], out_hbm.at[pl.ds(i, 1)], sem)
        cp.start()
        copies.append(cp)
    for cp in copies:
        cp.wait()


def kernel(x, index1, index2, index3):
    xflat = x.reshape(-1, _D)
    idx_packed = jnp.zeros((_NIDX,), jnp.int32)
    idx_packed = lax.dynamic_update_slice(idx_packed, index1.reshape(4), (0,))
    idx_packed = lax.dynamic_update_slice(idx_packed, index2.reshape(3), (_O2,))
    idx_packed = lax.dynamic_update_slice(idx_packed, index3.reshape(_NROWS), (_O3,))
    mesh = plsc.ScalarSubcoreMesh(axis_name="c", num_cores=1)
    out = pl.kernel(
        _body,
        out_type=jax.ShapeDtypeStruct((_NROWS, _D), jnp.float32),
        mesh=mesh,
        compiler_params=pltpu.CompilerParams(needs_layout_passes=False),
        scratch_types=[
            pltpu.SMEM((_NIDX,), jnp.int32),
            pltpu.SemaphoreType.DMA,
        ],
    )(idx_packed, xflat)
    return out.reshape(4, 3, _D)


# speedup vs baseline: 1.0761x; 1.0402x over previous
import jax
import jax.numpy as jnp
from jax import lax
from jax.experimental import pallas as pl
from jax.experimental.pallas import tpu as pltpu
from jax.experimental.pallas import tpu_sc as plsc

_D = 128
_NROWS = 12
_S1 = 128 * 64
_S2 = 64


def _body(i1_hbm, i2_hbm, i3_hbm, xflat_hbm, out_hbm, i1_s, i2_s, i3_s, sem):
    c1 = pltpu.make_async_copy(i1_hbm, i1_s, sem)
    c2 = pltpu.make_async_copy(i2_hbm, i2_s, sem)
    c3 = pltpu.make_async_copy(i3_hbm, i3_s, sem)
    c1.start()
    c2.start()
    c3.start()
    c1.wait()
    c2.wait()
    c3.wait()
    copies = []
    for i in range(_NROWS):
        flat = i1_s[i // 3] * _S1 + i2_s[i % 3] * _S2 + i3_s[i]
        cp = pltpu.make_async_copy(
            xflat_hbm.at[pl.ds(flat, 1)], out_hbm.at[pl.ds(i, 1)], sem)
        cp.start()
        copies.append(cp)
    for cp in copies:
        cp.wait()


def kernel(x, index1, index2, index3):
    xflat = x.reshape(-1, _D)
    mesh = plsc.ScalarSubcoreMesh(axis_name="c", num_cores=1)
    out = pl.kernel(
        _body,
        out_type=jax.ShapeDtypeStruct((_NROWS, _D), jnp.float32),
        mesh=mesh,
        compiler_params=pltpu.CompilerParams(needs_layout_passes=False),
        scratch_types=[
            pltpu.SMEM((4,), jnp.int32),
            pltpu.SMEM((3,), jnp.int32),
            pltpu.SMEM((_NROWS,), jnp.int32),
            pltpu.SemaphoreType.DMA,
        ],
    )(index1.reshape(4), index2.reshape(3), index3.reshape(_NROWS), xflat)
    return out.reshape(4, 3, _D)
